# baseline (speedup 1.0000x reference)
import functools

import jax
import jax.numpy as jnp
from jax import lax
from jax.experimental import pallas as pl
from jax.experimental.pallas import tpu as pltpu
from jax.experimental.pallas import tpu_sc as plsc

D = 128
H = 50
HP = 56
NC = 2
NS = 16
NW = NC * NS
NBUF = 8
LA = 4


@functools.lru_cache(maxsize=None)
def _build(batch):
  assert batch % NW == 0
  rows_per_w = batch // NW
  assert rows_per_w % NBUF == 0 and rows_per_w > NBUF
  mesh = plsc.VectorSubcoreMesh(core_axis_name="c", subcore_axis_name="s")

  @functools.partial(
      pl.kernel,
      out_type=jax.ShapeDtypeStruct((batch, HP, D), jnp.float32),
      mesh=mesh,
      scratch_types=[
          pltpu.VMEM((rows_per_w, HP), jnp.int32),
          pltpu.VMEM((NBUF, HP, D), jnp.float32),
      ] + [pltpu.SemaphoreType.DMA] * (2 * NBUF),
  )
  def gather_kernel(table_hbm, ids_hbm, out_hbm, idx_v, rows_v, *sems):
    gsem = sems[:NBUF]
    ssem = sems[NBUF:]
    wid = lax.axis_index("s") * NC + lax.axis_index("c")
    base = wid * rows_per_w

    pltpu.sync_copy(ids_hbm.at[pl.ds(base, rows_per_w)], idx_v)

    def start_gather(c, slot):
      pltpu.async_copy(
          table_hbm.at[idx_v.at[c]], rows_v.at[slot], gsem[slot])

    def wait_gather(slot):
      pltpu.make_async_copy(
          out_hbm.at[0], rows_v.at[slot], gsem[slot]).wait()

    def wait_store(slot):
      pltpu.make_async_copy(
          rows_v.at[slot], out_hbm.at[0], ssem[slot]).wait()

    for c in range(LA):
      start_gather(c, c % NBUF)

    def outer(g, carry):
      c0 = g * NBUF
      for b in range(NBUF):
        c = c0 + b
        sg = (b + LA) % NBUF

        @pl.when(c + LA < rows_per_w)
        def _():
          @pl.when(c >= NBUF - LA)
          def _():
            wait_store(sg)
          start_gather(c + LA, sg)

        wait_gather(b)
        pltpu.async_copy(rows_v.at[b], out_hbm.at[base + c], ssem[b])
      return carry

    lax.fori_loop(0, rows_per_w // NBUF, outer, 0)

    for b in range(NBUF):
      wait_store(b)

  return gather_kernel


def kernel(ids, embedding_matrix):
  b, h = ids.shape
  vocab, d = embedding_matrix.shape
  assert h == H and d == D
  ids_pad = jnp.pad(ids.astype(jnp.int32), ((0, 0), (0, HP - H)))
  out = _build(b)(embedding_matrix, ids_pad)
  return out[:, :H, :]


# speedup vs baseline: 16.5371x; 16.5371x over previous
import functools

import jax
import jax.numpy as jnp
from jax import lax
from jax.experimental import pallas as pl
from jax.experimental.pallas import tpu as pltpu
from jax.experimental.pallas import tpu_sc as plsc

D = 128
H = 50
CHUNK = 64
NC = 2
NS = 16
NW = NC * NS
NBUF = 5
LA = 3
L = 16


@functools.lru_cache(maxsize=None)
def _build(batch):
  assert batch % (NW * CHUNK) == 0
  br = batch // NW
  sb_per_h = br // CHUNK
  n_chunks = H * sb_per_h
  assert n_chunks % NBUF == 0
  mesh = plsc.VectorSubcoreMesh(core_axis_name="c", subcore_axis_name="s")

  @functools.partial(
      pl.kernel,
      out_type=jax.ShapeDtypeStruct((H * batch, D), jnp.float32),
      mesh=mesh,
      scratch_types=[
          pltpu.VMEM((br, H), jnp.int32),
          pltpu.VMEM((NBUF, CHUNK), jnp.int32),
          pltpu.VMEM((NBUF, CHUNK, D), jnp.float32),
      ] + [pltpu.SemaphoreType.DMA] * (2 * NBUF),
      compiler_params=pltpu.CompilerParams(needs_layout_passes=False),
  )
  def gather_kernel(table_hbm, ids_hbm, out_hbm, idx_slab, idx_buf, rows_v,
                    *sems):
    gsem = sems[:NBUF]
    ssem = sems[NBUF:]
    wid = lax.axis_index("s") * NC + lax.axis_index("c")
    base_b = wid * br

    pltpu.sync_copy(ids_hbm.at[pl.ds(base_b, br)], idx_slab)

    def start_gather(c, slot):
      h = c // sb_per_h
      row0 = (c % sb_per_h) * CHUNK
      for g in range(CHUNK // L):
        rows = lax.iota(jnp.int32, L) + (row0 + g * L)
        cols = jnp.full((L,), 0, jnp.int32) + h
        vals = plsc.load_gather(idx_slab, [rows, cols])
        idx_buf[slot, pl.ds(g * L, L)] = vals
      pltpu.async_copy(
          table_hbm.at[idx_buf.at[slot]], rows_v.at[slot], gsem[slot])

    def wait_gather(slot):
      pltpu.make_async_copy(
          out_hbm.at[pl.ds(0, CHUNK)], rows_v.at[slot], gsem[slot]).wait()

    def wait_store(slot):
      pltpu.make_async_copy(
          rows_v.at[slot], out_hbm.at[pl.ds(0, CHUNK)], ssem[slot]).wait()

    def out_row(c):
      h = c // sb_per_h
      row0 = (c % sb_per_h) * CHUNK
      return h * batch + base_b + row0

    for c in range(LA):
      start_gather(c, c % NBUF)

    def outer(g, carry):
      c0 = g * NBUF
      for b in range(NBUF):
        c = c0 + b
        sg = (b + LA) % NBUF

        @pl.when(c + LA < n_chunks)
        def _():
          @pl.when(c >= NBUF - LA)
          def _():
            wait_store(sg)
          start_gather(c + LA, sg)

        wait_gather(b)
        pltpu.async_copy(
            rows_v.at[b], out_hbm.at[pl.ds(out_row(c), CHUNK)], ssem[b])
      return carry

    lax.fori_loop(0, n_chunks // NBUF, outer, 0)

    for b in range(NBUF):
      wait_store(b)

  return gather_kernel


def kernel(ids, embedding_matrix):
  b, h = ids.shape
  vocab, d = embedding_matrix.shape
  assert h == H and d == D
  out = _build(b)(embedding_matrix, ids.astype(jnp.int32))
  return out.reshape(H, b, D).transpose(1, 0, 2)
